# baseline (speedup 1.0000x reference)
import jax
import jax.numpy as jnp
from jax.experimental import pallas as pl
from jax.experimental.pallas import tpu as pltpu


def _mm_body(p_ref, w_ref, z_ref):
    z_ref[0] = jnp.dot(p_ref[...], w_ref[0],
                       preferred_element_type=jnp.float32)


def kernel(x, trans_row, trans_col, trans_val, spiral_idx, W, b):
    B, N_IN, C = x.shape
    N_OUT, S = spiral_idx.shape
    C_OUT = W.shape[1]

    gathered = jnp.take(x, trans_col, axis=1) * trans_val[None, :, None]
    pooled = jnp.zeros((B, N_OUT, C), x.dtype).at[:, trans_row, :].add(gathered)

    pr = pooled.reshape(B * N_OUT, C)
    Ws = W.reshape(S, C, C_OUT)

    TILE = 512
    n_tiles = (B * N_OUT) // TILE
    Z = pl.pallas_call(
        _mm_body,
        grid=(n_tiles, S),
        in_specs=[
            pl.BlockSpec((TILE, C), lambda r, s: (r, 0)),
            pl.BlockSpec((1, C, C_OUT), lambda r, s: (s, 0, 0)),
        ],
        out_specs=pl.BlockSpec((1, TILE, C_OUT), lambda r, s: (s, r, 0)),
        out_shape=jax.ShapeDtypeStruct((S, B * N_OUT, C_OUT), jnp.float32),
    )(pr, Ws)

    Z = Z.reshape(S, B, N_OUT, C_OUT)
    acc = b.astype(jnp.float32)[None, None, :]
    out = jnp.zeros((B, N_OUT, C_OUT), jnp.float32) + acc
    for s in range(S):
        out = out + jnp.take(Z[s], spiral_idx[:, s], axis=1)
    return jax.nn.relu(out)


# speedup vs baseline: 1.0635x; 1.0635x over previous
import functools

import jax
import jax.numpy as jnp
from jax import lax
from jax.experimental import pallas as pl
from jax.experimental.pallas import tpu as pltpu
from jax.experimental.pallas import tpu_sc as plsc


def _mm_body(p_ref, w_ref, z_ref):
    s = pl.program_id(2)
    z_ref[0] = jnp.dot(p_ref[0], w_ref[s],
                       preferred_element_type=jnp.float32)


def _matmul_z(pooled, Ws):
    B, N_OUT, C = pooled.shape
    S, _, C_OUT = Ws.shape
    TILE = 400
    return pl.pallas_call(
        _mm_body,
        grid=(B, N_OUT // TILE, S),
        in_specs=[
            pl.BlockSpec((1, TILE, C), lambda b, u, s: (b, u, 0)),
            pl.BlockSpec((S, C, C_OUT), lambda b, u, s: (0, 0, 0)),
        ],
        out_specs=pl.BlockSpec((1, TILE, C_OUT),
                               lambda b, u, s: (s, u, b)),
        out_shape=jax.ShapeDtypeStruct((S, N_OUT, B * C_OUT), jnp.float32),
    )(pooled, Ws)


def _make_sc_gather(S, N_PAD, ROW, NW, VPW, VC, NTI):
    NCH = VPW // VC
    T = NCH * S
    NJ = ROW // 16
    mesh = plsc.VectorSubcoreMesh(core_axis_name="c", subcore_axis_name="s")

    @functools.partial(
        pl.kernel, mesh=mesh,
        out_type=jax.ShapeDtypeStruct((N_PAD, ROW), jnp.float32),
        scratch_types=[
            pltpu.VMEM((NTI, 8, 128), jnp.int32),
            pltpu.VMEM((2, VC, ROW), jnp.float32),
            pltpu.VMEM((VC, ROW), jnp.float32),
            pltpu.VMEM((1, ROW), jnp.float32),
            pltpu.SemaphoreType.DMA((2,)),
        ],
    )
    def sc_gather(z_hbm, idx_hbm, bias_hbm, out_hbm,
                  idx_v, zbuf, acc, bias_v, sem):
        wid = lax.axis_index("s") * 2 + lax.axis_index("c")
        pltpu.sync_copy(idx_hbm.at[wid], idx_v)
        pltpu.sync_copy(bias_hbm, bias_v)

        def fire(t):
            ireg = idx_v[t // 64, (t % 64) // 8, pl.ds((t % 8) * 16, 16)]
            pltpu.async_copy(z_hbm.at[ireg],
                             zbuf.at[t % 2], sem.at[t % 2])

        fire(0)

        def step(t, _):
            c, s = t // S, t % S
            pb = t % 2
            pltpu.make_async_copy(z_hbm.at[pl.ds(0, VC)],
                                  zbuf.at[pb], sem.at[pb]).wait()

            @pl.when(t + 1 < T)
            def _():
                fire(t + 1)

            @pl.when(s == 0)
            def _():
                def cp(j, _):
                    o = j * 16
                    for i in range(VC):
                        acc[i, pl.ds(o, 16)] = zbuf[pb, i, pl.ds(o, 16)]
                    return 0
                lax.fori_loop(0, NJ, cp, 0)

            @pl.when(s > 0)
            def _():
                def ad(j, _):
                    o = j * 16
                    for i in range(VC):
                        plsc.addupdate(acc.at[i, pl.ds(o, 16)],
                                       zbuf[pb, i, pl.ds(o, 16)])
                    return 0
                lax.fori_loop(0, NJ, ad, 0)

            @pl.when(s == S - 1)
            def _():
                def fin(j, _):
                    o = j * 16
                    bv = bias_v[0, pl.ds(o, 16)]
                    for i in range(VC):
                        acc[i, pl.ds(o, 16)] = jnp.maximum(
                            acc[i, pl.ds(o, 16)] + bv, 0.0)
                    return 0
                lax.fori_loop(0, NJ, fin, 0)
                pltpu.sync_copy(acc,
                                out_hbm.at[pl.ds(wid * VPW + c * VC, VC)])
            return 0

        lax.fori_loop(0, T, step, 0)

    return sc_gather


def kernel(x, trans_row, trans_col, trans_val, spiral_idx, W, b):
    B, N_IN, C = x.shape
    N_OUT, S = spiral_idx.shape
    C_OUT = W.shape[1]
    ROW = B * C_OUT
    NW = 32
    VC = 16
    VPW = -(-N_OUT // (NW * VC)) * VC
    N_PAD = NW * VPW
    NCH = VPW // VC
    T = NCH * S
    T_PAD = -(-T // 64) * 64
    NTI = T_PAD // 64

    gathered = jnp.take(x, trans_col, axis=1) * trans_val[None, :, None]
    pooled = jnp.zeros((B, N_OUT, C), x.dtype).at[:, trans_row, :].add(gathered)

    Ws = W.reshape(S, C, C_OUT)
    Z = _matmul_z(pooled, Ws)
    z2d = Z.reshape(S * N_OUT, ROW)

    sp = jnp.pad(spiral_idx.astype(jnp.int32), ((0, N_PAD - N_OUT), (0, 0)))
    idx_flat = sp.T + jnp.arange(S, dtype=jnp.int32)[:, None] * N_OUT
    idx_wp = (idx_flat.reshape(S, NW, NCH, VC)
              .transpose(1, 2, 0, 3)
              .reshape(NW, T, VC))
    idx_wp = jnp.pad(idx_wp, ((0, 0), (0, T_PAD - T), (0, 0)))
    idx_wp = idx_wp.reshape(NW, NTI, 8, 128)
    bias_row = jnp.tile(b.astype(jnp.float32), B)[None, :]

    sc = _make_sc_gather(S, N_PAD, ROW, NW, VPW, VC, NTI)
    out_t = sc(z2d, idx_wp, bias_row)
    return out_t[:N_OUT].reshape(N_OUT, B, C_OUT).transpose(1, 0, 2)


# speedup vs baseline: 1.5850x; 1.4905x over previous
import functools

import jax
import jax.numpy as jnp
from jax import lax
from jax.experimental import pallas as pl
from jax.experimental.pallas import tpu as pltpu
from jax.experimental.pallas import tpu_sc as plsc


def _mm_body(p_ref, w_ref, z_ref):
    s = pl.program_id(2)
    z_ref[0] = jnp.dot(p_ref[0], w_ref[s],
                       preferred_element_type=jnp.float32)


def _matmul_z(pooled, Ws):
    B, N_OUT, C = pooled.shape
    S, _, C_OUT = Ws.shape
    TILE = 400
    return pl.pallas_call(
        _mm_body,
        grid=(B, N_OUT // TILE, S),
        in_specs=[
            pl.BlockSpec((1, TILE, C), lambda b, u, s: (b, u, 0)),
            pl.BlockSpec((S, C, C_OUT), lambda b, u, s: (0, 0, 0)),
        ],
        out_specs=pl.BlockSpec((1, TILE, C_OUT),
                               lambda b, u, s: (s, u, b)),
        out_shape=jax.ShapeDtypeStruct((S, N_OUT, B * C_OUT), jnp.float32),
    )(pooled, Ws)


def _make_sc_pool(B, N_IN, C, NNZP, NH, NW_SC):
    HALF = 10000
    SLAB = 10240
    STRIPE = SLAB // 16
    CH = 128
    NCHUNK = NNZP // (16 * CH)
    NSLAB = B * NH // 2
    mesh = plsc.VectorSubcoreMesh(core_axis_name="c", subcore_axis_name="s")

    @functools.partial(
        pl.kernel, mesh=mesh,
        out_type=jax.ShapeDtypeStruct((B, NH, HALF, C), jnp.float32),
        scratch_types=[
            pltpu.VMEM((128,), jnp.int32),
            pltpu.VMEM((128,), jnp.int32),
            pltpu.VMEM((128,), jnp.float32),
            pltpu.VMEM((2, 128,), jnp.int32),
            pltpu.VMEM((128,), jnp.int32),
            pltpu.VMEM((2, CH, C), jnp.float32),
            pltpu.VMEM((64, C), jnp.float32),
            pltpu.VMEM_SHARED((SLAB, C), jnp.float32),
            pltpu.SemaphoreType.DMA((2,)),
        ],
    )
    def sc_pool(x_hbm, col_hbm, row_hbm, val_hbm, pooled_hbm,
                col_v, row_v, val_v, gidx_v, ridx_v, xbuf, zbuf, slab, sem):
        core = lax.axis_index("c")
        tid = lax.axis_index("s")

        def z16(j, _):
            zbuf[j // (C // 16), pl.ds((j % (C // 16)) * 16, 16)] = (
                jnp.zeros((16,), jnp.float32))
            return 0
        lax.fori_loop(0, (64 * C) // 16, z16, 0)

        def slab_body(sl, _):
            b = core * (B // 2) + sl // NH
            h = sl % NH
            def zc(i, _):
                pltpu.sync_copy(
                    zbuf, slab.at[pl.ds(tid * STRIPE + i * 64, 64)])
                return 0
            lax.fori_loop(0, STRIPE // 64, zc, 0)
            plsc.subcore_barrier()

            xoff = b * N_IN
            roff = h * HALF

            def fire(c):
                g = tid * NCHUNK + c
                base = g * CH
                pb = c % 2
                pltpu.sync_copy(col_hbm.at[pl.ds(base, CH)], col_v)

                def gi(k, _):
                    gidx_v[pb, pl.ds(k * 16, 16)] = (
                        col_v[pl.ds(k * 16, 16)] + xoff)
                    return 0
                lax.fori_loop(0, CH // 16, gi, 0)
                pltpu.async_copy(x_hbm.at[gidx_v.at[pb]],
                                 xbuf.at[pb], sem.at[pb])

            fire(0)

            def chunk_body(c, _):
                pb = c % 2
                g = tid * NCHUNK + c
                base = g * CH
                pltpu.sync_copy(row_hbm.at[pl.ds(base, CH)], row_v)
                pltpu.sync_copy(val_hbm.at[pl.ds(base, CH)], val_v)
                pltpu.make_async_copy(x_hbm.at[pl.ds(0, CH)],
                                      xbuf.at[pb], sem.at[pb]).wait()

                @pl.when(c + 1 < NCHUNK)
                def _():
                    fire(c + 1)

                def ri(k, _):
                    rr = row_v[pl.ds(k * 16, 16)] - roff
                    ok = (rr >= 0) & (rr < HALF)
                    ridx_v[pl.ds(k * 16, 16)] = jnp.where(ok, rr, HALF)
                    return 0
                lax.fori_loop(0, CH // 16, ri, 0)

                def scale_grp(grp, _):
                    vv = val_v[pl.ds(grp * 16, 16)]
                    for l in range(16):
                        i = grp * 16 + l
                        v = vv[l]
                        for j in range(C // 16):
                            xbuf[pb, i, pl.ds(j * 16, 16)] = (
                                xbuf[pb, i, pl.ds(j * 16, 16)] * v)
                    return 0
                lax.fori_loop(0, CH // 16, scale_grp, 0)

                pltpu.sync_copy(xbuf.at[pb], slab.at[ridx_v], add=True)
                return 0

            lax.fori_loop(0, NCHUNK, chunk_body, 0)
            plsc.subcore_barrier()

            @pl.when(tid * STRIPE < HALF - STRIPE)
            def _():
                pltpu.sync_copy(
                    slab.at[pl.ds(tid * STRIPE, STRIPE)],
                    pooled_hbm.at[b, h, pl.ds(tid * STRIPE, STRIPE)])

            @pl.when(tid * STRIPE >= HALF - STRIPE)
            def _():
                @pl.when(tid * STRIPE < HALF)
                def _():
                    pltpu.sync_copy(
                        slab.at[pl.ds(tid * STRIPE, HALF - 15 * STRIPE)],
                        pooled_hbm.at[b, h,
                                      pl.ds(tid * STRIPE, HALF - 15 * STRIPE)])
            return 0

        lax.fori_loop(0, NSLAB, slab_body, 0)

    return sc_pool


def _make_sc_gather(S, N_PAD, ROW, NW, VPW, VC, NTI):
    NCH = VPW // VC
    T = NCH * S
    NJ = ROW // 16
    mesh = plsc.VectorSubcoreMesh(core_axis_name="c", subcore_axis_name="s")

    @functools.partial(
        pl.kernel, mesh=mesh,
        out_type=jax.ShapeDtypeStruct((N_PAD, ROW), jnp.float32),
        scratch_types=[
            pltpu.VMEM((NTI, 8, 128), jnp.int32),
            pltpu.VMEM((2, VC, ROW), jnp.float32),
            pltpu.VMEM((VC, ROW), jnp.float32),
            pltpu.VMEM((1, ROW), jnp.float32),
            pltpu.SemaphoreType.DMA((2,)),
        ],
    )
    def sc_gather(z_hbm, idx_hbm, bias_hbm, out_hbm,
                  idx_v, zbuf, acc, bias_v, sem):
        wid = lax.axis_index("s") * 2 + lax.axis_index("c")
        pltpu.sync_copy(idx_hbm.at[wid], idx_v)
        pltpu.sync_copy(bias_hbm, bias_v)

        def fire(t):
            ireg = idx_v[t // 64, (t % 64) // 8, pl.ds((t % 8) * 16, 16)]
            pltpu.async_copy(z_hbm.at[ireg],
                             zbuf.at[t % 2], sem.at[t % 2])

        fire(0)

        def step(t, _):
            c, s = t // S, t % S
            pb = t % 2
            pltpu.make_async_copy(z_hbm.at[pl.ds(0, VC)],
                                  zbuf.at[pb], sem.at[pb]).wait()

            @pl.when(t + 1 < T)
            def _():
                fire(t + 1)

            @pl.when(s == 0)
            def _():
                def cp(j, _):
                    o = j * 16
                    for i in range(VC):
                        acc[i, pl.ds(o, 16)] = zbuf[pb, i, pl.ds(o, 16)]
                    return 0
                lax.fori_loop(0, NJ, cp, 0)

            @pl.when(s > 0)
            def _():
                def ad(j, _):
                    o = j * 16
                    for i in range(VC):
                        plsc.addupdate(acc.at[i, pl.ds(o, 16)],
                                       zbuf[pb, i, pl.ds(o, 16)])
                    return 0
                lax.fori_loop(0, NJ, ad, 0)

            @pl.when(s == S - 1)
            def _():
                def fin(j, _):
                    o = j * 16
                    bv = bias_v[0, pl.ds(o, 16)]
                    for i in range(VC):
                        acc[i, pl.ds(o, 16)] = jnp.maximum(
                            acc[i, pl.ds(o, 16)] + bv, 0.0)
                    return 0
                lax.fori_loop(0, NJ, fin, 0)
                pltpu.sync_copy(acc,
                                out_hbm.at[pl.ds(wid * VPW + c * VC, VC)])
            return 0

        lax.fori_loop(0, T, step, 0)

    return sc_gather


def kernel(x, trans_row, trans_col, trans_val, spiral_idx, W, b):
    B, N_IN, C = x.shape
    N_OUT, S = spiral_idx.shape
    C_OUT = W.shape[1]
    ROW = B * C_OUT
    NW = 32
    VC = 16
    VPW = -(-N_OUT // (NW * VC)) * VC
    N_PAD = NW * VPW
    NCH = VPW // VC
    T = NCH * S
    T_PAD = -(-T // 64) * 64
    NTI = T_PAD // 64

    NNZ = trans_row.shape[0]
    NNZP = -(-NNZ // 2048) * 2048
    NH = 2
    pad = NNZP - NNZ
    colp = jnp.pad(trans_col.astype(jnp.int32), (0, pad))
    rowp = jnp.pad(trans_row.astype(jnp.int32), (0, pad))
    valp = jnp.pad(trans_val.astype(jnp.float32), (0, pad))
    x2d = x.reshape(B * N_IN, C)
    pool = _make_sc_pool(B, N_IN, C, NNZP, NH, 32)
    pooled = pool(x2d, colp, rowp, valp).reshape(B, N_OUT, C)

    Ws = W.reshape(S, C, C_OUT)
    Z = _matmul_z(pooled, Ws)
    z2d = Z.reshape(S * N_OUT, ROW)

    sp = jnp.pad(spiral_idx.astype(jnp.int32), ((0, N_PAD - N_OUT), (0, 0)))
    idx_flat = sp.T + jnp.arange(S, dtype=jnp.int32)[:, None] * N_OUT
    idx_wp = (idx_flat.reshape(S, NW, NCH, VC)
              .transpose(1, 2, 0, 3)
              .reshape(NW, T, VC))
    idx_wp = jnp.pad(idx_wp, ((0, 0), (0, T_PAD - T), (0, 0)))
    idx_wp = idx_wp.reshape(NW, NTI, 8, 128)
    bias_row = jnp.tile(b.astype(jnp.float32), B)[None, :]

    sc = _make_sc_gather(S, N_PAD, ROW, NW, VPW, VC, NTI)
    out_t = sc(z2d, idx_wp, bias_row)
    return out_t[:N_OUT].reshape(N_OUT, B, C_OUT).transpose(1, 0, 2)
